# baseline (speedup 1.0000x reference)
import functools

import jax
import jax.numpy as jnp
from jax import lax
from jax.experimental import pallas as pl
from jax.experimental.pallas import tpu as pltpu
from jax.experimental.pallas import tpu_sc as plsc

N = 100000
E = 3200000
NPAD = 100096
SENT = N
NC = 2
NS = 16
NTILES = NC * NS
CH = 64
EPAD = 3211264
EROWS = EPAD // CH
CPT = EPAD // NTILES // CH
HB = 7
RING = 2 * HB
BR = 112
NBIG = CPT // BR
NSTEP = BR // HB
OROWS = NPAD // NS

_mesh = plsc.VectorSubcoreMesh(
    core_axis_name="c", subcore_axis_name="s", num_cores=NC, num_subcores=NS)


def _make_prop(w):
  @functools.partial(
      pl.kernel,
      out_type=jax.ShapeDtypeStruct((NC, NPAD, w), jnp.float32),
      mesh=_mesh,
      compiler_params=pltpu.CompilerParams(use_tc_tiling_on_sc=False),
      scratch_types=[
          pltpu.VMEM((2, BR, CH), jnp.int32),
          pltpu.VMEM((RING, CH, w), jnp.float32),
          pltpu.VMEM_SHARED((NPAD, w), jnp.float32),
          pltpu.SemaphoreType.DMA((RING,)),
          pltpu.SemaphoreType.DMA((RING,)),
      ],
  )
  def prop(x_hbm, src_hbm, dst_hbm, zeros_hbm, out_hbm,
           idx_v, rows_v, acc_sh, gsem, ssem):
    cid = lax.axis_index("c")
    sid = lax.axis_index("s")
    wid = cid * NS + sid
    pltpu.sync_copy(zeros_hbm.at[pl.ds(sid * OROWS, OROWS)],
                    acc_sh.at[pl.ds(sid * OROWS, OROWS)])
    plsc.subcore_barrier()
    row0 = wid * CPT

    def gather(c, b):
      pltpu.async_copy(x_hbm.at[idx_v.at[0, c]], rows_v.at[b], gsem.at[b])

    def scatter(c, b):
      pltpu.make_async_copy(x_hbm.at[idx_v.at[0, 0]], rows_v.at[b],
                            gsem.at[b]).wait()
      pltpu.async_copy(rows_v.at[b], acc_sh.at[idx_v.at[1, c]],
                       ssem.at[b], add=True)

    def swait(b):
      pltpu.make_async_copy(rows_v.at[b], acc_sh.at[idx_v.at[1, 0]],
                            ssem.at[b]).wait()

    def big_body(big, carry):
      r0 = row0 + big * BR
      pltpu.sync_copy(src_hbm.at[pl.ds(r0, BR)], idx_v.at[0])
      pltpu.sync_copy(dst_hbm.at[pl.ds(r0, BR)], idx_v.at[1])
      for k in range(HB):
        gather(k, k)

      def pair_body(sp, c2):
        s0 = sp * 2 * HB
        for k in range(HB):
          scatter(s0 + k, k)
        for k in range(HB):
          @pl.when(sp > 0)
          def _():
            swait(HB + k)
          gather(s0 + HB + k, HB + k)
        for k in range(HB):
          scatter(s0 + HB + k, HB + k)
        for k in range(HB):
          swait(k)
          gather(s0 + 2 * HB + k, k)
        return c2

      lax.fori_loop(0, NSTEP // 2 - 1, pair_body, 0)
      s14 = (NSTEP - 2) * HB
      for k in range(HB):
        scatter(s14 + k, k)
      for k in range(HB):
        swait(HB + k)
        gather(s14 + HB + k, HB + k)
      for k in range(HB):
        scatter(s14 + HB + k, HB + k)
      for b in range(RING):
        swait(b)
      return carry

    lax.fori_loop(0, NBIG, big_body, 0)
    plsc.subcore_barrier()
    pltpu.sync_copy(acc_sh.at[pl.ds(sid * OROWS, OROWS)],
                    out_hbm.at[cid, pl.ds(sid * OROWS, OROWS)])

  return prop


@functools.partial(
    pl.kernel,
    out_type=jax.ShapeDtypeStruct((NC, 2, NPAD, 8), jnp.float32),
    mesh=_mesh,
    compiler_params=pltpu.CompilerParams(use_tc_tiling_on_sc=False),
    scratch_types=[
        pltpu.VMEM((2, BR, CH), jnp.int32),
        pltpu.VMEM((CH, 8), jnp.float32),
        pltpu.VMEM_SHARED((NPAD, 8), jnp.float32),
        pltpu.VMEM_SHARED((NPAD, 8), jnp.float32),
        pltpu.SemaphoreType.DMA((HB,)),
        pltpu.SemaphoreType.DMA((HB,)),
    ],
)
def _degrees(src_hbm, dst_hbm, ones_hbm, zeros_hbm, out_hbm,
             idx_v, ones_v, accin_sh, accout_sh, isem, osem):
  cid = lax.axis_index("c")
  sid = lax.axis_index("s")
  wid = cid * NS + sid
  pltpu.sync_copy(ones_hbm, ones_v)
  pltpu.sync_copy(zeros_hbm.at[pl.ds(sid * OROWS, OROWS)],
                  accin_sh.at[pl.ds(sid * OROWS, OROWS)])
  pltpu.sync_copy(zeros_hbm.at[pl.ds(sid * OROWS, OROWS)],
                  accout_sh.at[pl.ds(sid * OROWS, OROWS)])
  plsc.subcore_barrier()
  row0 = wid * CPT

  def big_body(big, carry):
    r0 = row0 + big * BR
    pltpu.sync_copy(src_hbm.at[pl.ds(r0, BR)], idx_v.at[0])
    pltpu.sync_copy(dst_hbm.at[pl.ds(r0, BR)], idx_v.at[1])

    def grp_body(g, c):
      for b in range(HB):
        r = g * HB + b
        pltpu.async_copy(ones_v, accin_sh.at[idx_v.at[1, r]],
                         isem.at[b], add=True)
        pltpu.async_copy(ones_v, accout_sh.at[idx_v.at[0, r]],
                         osem.at[b], add=True)
      for b in range(HB):
        pltpu.make_async_copy(ones_v, accin_sh.at[idx_v.at[1, 0]],
                              isem.at[b]).wait()
        pltpu.make_async_copy(ones_v, accout_sh.at[idx_v.at[0, 0]],
                              osem.at[b]).wait()
      return c

    return lax.fori_loop(0, BR // HB, grp_body, carry)

  lax.fori_loop(0, NBIG, big_body, 0)
  plsc.subcore_barrier()
  pltpu.sync_copy(accin_sh.at[pl.ds(sid * OROWS, OROWS)],
                  out_hbm.at[cid, 0, pl.ds(sid * OROWS, OROWS)])
  pltpu.sync_copy(accout_sh.at[pl.ds(sid * OROWS, OROWS)],
                  out_hbm.at[cid, 1, pl.ds(sid * OROWS, OROWS)])


_prop16 = _make_prop(16)
_prop8 = _make_prop(8)


NR = NPAD // 128
PR16 = NPAD * 16 // 128
PR8 = NPAD * 8 // 128


def _mask16(x):
  r = lax.broadcasted_iota(jnp.int32, x.shape, 0)
  l = lax.broadcasted_iota(jnp.int32, x.shape, 1)
  return jnp.where(8 * r + l // 16 < N, x, 0.0)


def _mask8(x):
  r = lax.broadcasted_iota(jnp.int32, x.shape, 0)
  l = lax.broadcasted_iota(jnp.int32, x.shape, 1)
  return jnp.where(16 * r + l // 8 < N, x, 0.0)


def _iota2(shape):
  a = lax.broadcasted_iota(jnp.int32, shape, 0)
  b = lax.broadcasted_iota(jnp.int32, shape, 1)
  return a, b


def _kron16(w):
  a, b = _iota2((128, 128))
  return jnp.where(a // 16 == b // 16, jnp.tile(w, (8, 8)), 0.0)


def _stage0(degp_ref, x1_ref, ndp_ref, nsp_ref, nd_ref):
  din = degp_ref[0, 0] + degp_ref[1, 0]
  dout = degp_ref[0, 1] + degp_ref[1, 1]
  p, l = _iota2((1024, 128))
  r8c0 = jnp.where((p // 8 == l) & (p % 8 == 0), 1.0, 0.0)
  def tonm(x):
    return jnp.dot(jnp.reshape(x, (NR, 1024)), r8c0,
                   preferred_element_type=jnp.float32)
  in_nm = tonm(din)
  out_nm = tonm(dout)
  ns = lax.rsqrt(jnp.maximum(out_nm, 1.0))
  nd = lax.rsqrt(jnp.maximum(in_nm, 1.0))
  l2, q = _iota2((128, 2048))
  e16 = jnp.where(q // 16 == l2, 1.0, 0.0)
  ndp_ref[...] = jnp.reshape(jnp.dot(nd, e16,
                                     preferred_element_type=jnp.float32),
                             (PR16, 128))
  nsp_ref[...] = jnp.reshape(jnp.dot(ns, e16,
                                     preferred_element_type=jnp.float32),
                             (PR16, 128))
  nd_ref[...] = nd
  r = lax.broadcasted_iota(jnp.int32, (NR, 128), 0)
  lm = lax.broadcasted_iota(jnp.int32, (NR, 128), 1)
  x1 = jnp.where(128 * r + lm < N, in_nm * ns, 0.0)
  l3, q8 = _iota2((128, 1024))
  e8c0 = jnp.where((q8 // 8 == l3) & (q8 % 8 == 0), 1.0, 0.0)
  x1_ref[...] = jnp.reshape(jnp.dot(x1, e8c0,
                                    preferred_element_type=jnp.float32),
                            (PR8, 128))


def _stage1(mp_ref, nd_ref, nsp_ref, w1_ref, b1_ref, x2_ref):
  m8 = mp_ref[0] + mp_ref[1]
  p, l = _iota2((1024, 128))
  r8c0 = jnp.where((p // 8 == l) & (p % 8 == 0), 1.0, 0.0)
  m_nm = jnp.dot(jnp.reshape(m8, (NR, 1024)), r8c0,
                 preferred_element_type=jnp.float32) * nd_ref[...]
  l2, q = _iota2((128, 2048))
  e16w1 = jnp.where(q // 16 == l2, 1.0, 0.0) * jnp.tile(w1_ref[...], (128, 128))
  h = jnp.reshape(jnp.dot(m_nm, e16w1, preferred_element_type=jnp.float32),
                  (PR16, 128))
  b1 = jnp.tile(b1_ref[...], (1, 8))
  x2_ref[...] = _mask16(jnp.maximum(h + b1, 0.0) * nsp_ref[...])


def _stage2(mp_ref, ndp_ref, nsp_ref, w2_ref, b2_ref, w3_ref, x3_ref):
  m = (mp_ref[0] + mp_ref[1]) * ndp_ref[...]
  b2 = jnp.tile(b2_ref[...], (1, 8))
  h = jnp.maximum(
      jnp.dot(m, _kron16(w2_ref[...]), preferred_element_type=jnp.float32)
      + b2, 0.0)
  x3_ref[...] = _mask16(
      jnp.dot(h, _kron16(w3_ref[...]), preferred_element_type=jnp.float32)
      * nsp_ref[...])


def _stage3(mp_ref, ndp_ref, nsp_ref, b3_ref, w4_ref, x4_ref):
  m = (mp_ref[0] + mp_ref[1]) * ndp_ref[...]
  b3 = jnp.tile(b3_ref[...], (1, 8))
  h = jnp.maximum(m + b3, 0.0)
  x4_ref[...] = _mask16(
      jnp.dot(h, _kron16(w4_ref[...]), preferred_element_type=jnp.float32)
      * nsp_ref[...])


def _stage4(mp_ref, ndp_ref, b4_ref, wc_ref, bc_ref, out_ref):
  m = (mp_ref[0] + mp_ref[1]) * ndp_ref[...]
  b4 = jnp.tile(b4_ref[...], (1, 8))
  h = _mask16(jnp.maximum(m + b4, 0.0))
  lane = jnp.sum(h, axis=0, keepdims=True)
  l, j = _iota2((128, 16))
  sel = jnp.where(l % 16 == j, 1.0, 0.0)
  g = jnp.dot(lane, sel, preferred_element_type=jnp.float32) * (1.0 / N)
  out_ref[...] = (
      jnp.dot(g[:, :4], wc_ref[...], preferred_element_type=jnp.float32)
      + bc_ref[...])


def _tc(body, out_shape, *args):
  return pl.pallas_call(body, out_shape=out_shape)(*args)


def _f32(*shape):
  return jax.ShapeDtypeStruct(shape, jnp.float32)


def kernel(edge_index, num_nodes, W1, b1, W2, b2, W3, b3, W4, b4, Wc, bc):
  del num_nodes
  pad = jnp.full((EPAD - E,), SENT, jnp.int32)
  src2 = jnp.concatenate([edge_index[0], pad]).reshape(EROWS, CH)
  dst2 = jnp.concatenate([edge_index[1], pad]).reshape(EROWS, CH)
  ones8 = jnp.ones((CH, 8), jnp.float32)
  z8 = jnp.zeros((NPAD, 8), jnp.float32)
  z16 = jnp.zeros((NPAD, 16), jnp.float32)

  degp = _degrees(src2, dst2, ones8, z8)
  x1p, ndp, nsp, nd = _tc(
      _stage0,
      (_f32(PR8, 128), _f32(PR16, 128), _f32(PR16, 128), _f32(NR, 128)),
      degp.reshape(NC, 2, PR8, 128))
  m1 = _prop8(x1p.reshape(NPAD, 8), src2, dst2, z8)
  x2p = _tc(_stage1, _f32(PR16, 128), m1.reshape(NC, PR8, 128), nd, nsp,
            W1, b1.reshape(1, 16))
  m2 = _prop16(x2p.reshape(NPAD, 16), src2, dst2, z16)
  w3p = jnp.concatenate([W3, jnp.zeros((16, 8), jnp.float32)], axis=1)
  x3p = _tc(_stage2, _f32(PR16, 128), m2.reshape(NC, PR16, 128), ndp, nsp,
            W2, b2.reshape(1, 16), w3p)
  m3 = _prop16(x3p.reshape(NPAD, 16), src2, dst2, z16)
  b3p = jnp.concatenate([b3, jnp.zeros((8,), jnp.float32)]).reshape(1, 16)
  w4p = jnp.zeros((16, 16), jnp.float32).at[:8, :4].set(W4)
  x4p = _tc(_stage3, _f32(PR16, 128), m3.reshape(NC, PR16, 128), ndp, nsp,
            b3p, w4p)
  m4 = _prop16(x4p.reshape(NPAD, 16), src2, dst2, z16)
  b4p = jnp.concatenate([b4, jnp.zeros((12,), jnp.float32)]).reshape(1, 16)
  out = _tc(_stage4, _f32(1, 10), m4.reshape(NC, PR16, 128), ndp,
            b4p, Wc, bc.reshape(1, 10))
  return out


# speedup vs baseline: 1.2085x; 1.2085x over previous
import functools

import jax
import jax.numpy as jnp
from jax import lax
from jax.experimental import pallas as pl
from jax.experimental.pallas import tpu as pltpu
from jax.experimental.pallas import tpu_sc as plsc

N = 100000
E = 3200000
NPAD = 100096
SENT = N
NC = 2
NS = 16
NTILES = NC * NS
CPT = 784
EROWS = NTILES * CPT
EPAD = EROWS * 128
HB = 7
RING = 2 * HB
BR = 56
NBIG = CPT // BR
NSTEP = BR // HB
OROWS = NPAD // NS

_mesh = plsc.VectorSubcoreMesh(
    core_axis_name="c", subcore_axis_name="s", num_cores=NC, num_subcores=NS)


def _make_prop_pipe(w):
  @functools.partial(
      pl.kernel,
      out_type=jax.ShapeDtypeStruct((NC, NPAD, w), jnp.float32),
      mesh=_mesh,
      compiler_params=pltpu.CompilerParams(use_tc_tiling_on_sc=False),
      scratch_types=[
          pltpu.VMEM((2, BR, 128), jnp.int32),
          pltpu.VMEM((RING, 128, w), jnp.float32),
          pltpu.VMEM_SHARED((NPAD, w), jnp.float32),
          pltpu.SemaphoreType.DMA((RING,)),
          pltpu.SemaphoreType.DMA((RING,)),
      ],
  )
  def prop(x_hbm, src_hbm, dst_hbm, zeros_hbm, out_hbm,
           idx_v, rows_v, acc_sh, gsem, ssem):
    cid = lax.axis_index("c")
    sid = lax.axis_index("s")
    wid = cid * NS + sid
    pltpu.sync_copy(zeros_hbm.at[pl.ds(sid * OROWS, OROWS)],
                    acc_sh.at[pl.ds(sid * OROWS, OROWS)])
    plsc.subcore_barrier()
    row0 = wid * CPT

    def gather(c, b):
      pltpu.async_copy(x_hbm.at[idx_v.at[0, c]], rows_v.at[b], gsem.at[b])

    def scatter(c, b):
      pltpu.make_async_copy(x_hbm.at[idx_v.at[0, 0]], rows_v.at[b],
                            gsem.at[b]).wait()
      pltpu.async_copy(rows_v.at[b], acc_sh.at[idx_v.at[1, c]],
                       ssem.at[b], add=True)

    def swait(b):
      pltpu.make_async_copy(rows_v.at[b], acc_sh.at[idx_v.at[1, 0]],
                            ssem.at[b]).wait()

    def big_body(big, carry):
      r0 = row0 + big * BR
      pltpu.sync_copy(src_hbm.at[pl.ds(r0, BR)], idx_v.at[0])
      pltpu.sync_copy(dst_hbm.at[pl.ds(r0, BR)], idx_v.at[1])
      for k in range(HB):
        gather(k, k)

      def pair_body(sp, c2):
        s0 = sp * 2 * HB
        for k in range(HB):
          scatter(s0 + k, k)
        for k in range(HB):
          @pl.when(sp > 0)
          def _():
            swait(HB + k)
          gather(s0 + HB + k, HB + k)
        for k in range(HB):
          scatter(s0 + HB + k, HB + k)
        for k in range(HB):
          swait(k)
          gather(s0 + 2 * HB + k, k)
        return c2

      lax.fori_loop(0, NSTEP // 2 - 1, pair_body, 0)
      st = (NSTEP - 2) * HB
      for k in range(HB):
        scatter(st + k, k)
      for k in range(HB):
        swait(HB + k)
        gather(st + HB + k, HB + k)
      for k in range(HB):
        scatter(st + HB + k, HB + k)
      for b in range(RING):
        swait(b)
      return carry

    lax.fori_loop(0, NBIG, big_body, 0)
    plsc.subcore_barrier()
    pltpu.sync_copy(acc_sh.at[pl.ds(sid * OROWS, OROWS)],
                    out_hbm.at[cid, pl.ds(sid * OROWS, OROWS)])

  return prop


def _make_prop_grp(w):
  @functools.partial(
      pl.kernel,
      out_type=jax.ShapeDtypeStruct((NC, NPAD, w), jnp.float32),
      mesh=_mesh,
      compiler_params=pltpu.CompilerParams(use_tc_tiling_on_sc=False),
      scratch_types=[
          pltpu.VMEM((2, BR, 128), jnp.int32),
          pltpu.VMEM((HB, 128, w), jnp.float32),
          pltpu.VMEM_SHARED((NPAD, w), jnp.float32),
          pltpu.SemaphoreType.DMA((HB,)),
          pltpu.SemaphoreType.DMA((HB,)),
      ],
  )
  def prop(x_hbm, src_hbm, dst_hbm, zeros_hbm, out_hbm,
           idx_v, rows_v, acc_sh, gsem, ssem):
    cid = lax.axis_index("c")
    sid = lax.axis_index("s")
    wid = cid * NS + sid
    pltpu.sync_copy(zeros_hbm.at[pl.ds(sid * OROWS, OROWS)],
                    acc_sh.at[pl.ds(sid * OROWS, OROWS)])
    plsc.subcore_barrier()
    row0 = wid * CPT

    def big_body(big, carry):
      r0 = row0 + big * BR
      pltpu.sync_copy(src_hbm.at[pl.ds(r0, BR)], idx_v.at[0])
      pltpu.sync_copy(dst_hbm.at[pl.ds(r0, BR)], idx_v.at[1])

      def grp_body(g, c):
        first = jnp.logical_and(big == 0, g == 0)
        for b in range(HB):
          @pl.when(jnp.logical_not(first))
          def _():
            pltpu.make_async_copy(rows_v.at[b],
                                  acc_sh.at[idx_v.at[1, 0]],
                                  ssem.at[b]).wait()
          pltpu.async_copy(x_hbm.at[idx_v.at[0, g * HB + b]],
                           rows_v.at[b], gsem.at[b])
        for b in range(HB):
          pltpu.make_async_copy(x_hbm.at[idx_v.at[0, 0]],
                                rows_v.at[b], gsem.at[b]).wait()
          pltpu.async_copy(rows_v.at[b],
                           acc_sh.at[idx_v.at[1, g * HB + b]],
                           ssem.at[b], add=True)
        return c

      return lax.fori_loop(0, BR // HB, grp_body, carry)

    lax.fori_loop(0, NBIG, big_body, 0)
    for b in range(HB):
      pltpu.make_async_copy(rows_v.at[b], acc_sh.at[idx_v.at[1, 0]],
                            ssem.at[b]).wait()
    plsc.subcore_barrier()
    pltpu.sync_copy(acc_sh.at[pl.ds(sid * OROWS, OROWS)],
                    out_hbm.at[cid, pl.ds(sid * OROWS, OROWS)])

  return prop


_prop16 = _make_prop_grp(16)
_prop8 = _make_prop_pipe(8)


@functools.partial(
    pl.kernel,
    out_type=jax.ShapeDtypeStruct((NC, 2, NPAD, 8), jnp.float32),
    mesh=_mesh,
    compiler_params=pltpu.CompilerParams(use_tc_tiling_on_sc=False),
    scratch_types=[
        pltpu.VMEM((2, BR, 128), jnp.int32),
        pltpu.VMEM((128, 8), jnp.float32),
        pltpu.VMEM_SHARED((NPAD, 8), jnp.float32),
        pltpu.VMEM_SHARED((NPAD, 8), jnp.float32),
        pltpu.SemaphoreType.DMA((HB,)),
        pltpu.SemaphoreType.DMA((HB,)),
    ],
)
def _degrees(src_hbm, dst_hbm, ones_hbm, zeros_hbm, out_hbm,
             idx_v, ones_v, accin_sh, accout_sh, isem, osem):
  cid = lax.axis_index("c")
  sid = lax.axis_index("s")
  wid = cid * NS + sid
  pltpu.sync_copy(ones_hbm, ones_v)
  pltpu.sync_copy(zeros_hbm.at[pl.ds(sid * OROWS, OROWS)],
                  accin_sh.at[pl.ds(sid * OROWS, OROWS)])
  pltpu.sync_copy(zeros_hbm.at[pl.ds(sid * OROWS, OROWS)],
                  accout_sh.at[pl.ds(sid * OROWS, OROWS)])
  plsc.subcore_barrier()
  row0 = wid * CPT

  def big_body(big, carry):
    r0 = row0 + big * BR
    pltpu.sync_copy(src_hbm.at[pl.ds(r0, BR)], idx_v.at[0])
    pltpu.sync_copy(dst_hbm.at[pl.ds(r0, BR)], idx_v.at[1])

    def grp_body(g, c):
      for b in range(HB):
        r = g * HB + b
        pltpu.async_copy(ones_v, accin_sh.at[idx_v.at[1, r]],
                         isem.at[b], add=True)
        pltpu.async_copy(ones_v, accout_sh.at[idx_v.at[0, r]],
                         osem.at[b], add=True)
      for b in range(HB):
        pltpu.make_async_copy(ones_v, accin_sh.at[idx_v.at[1, 0]],
                              isem.at[b]).wait()
        pltpu.make_async_copy(ones_v, accout_sh.at[idx_v.at[0, 0]],
                              osem.at[b]).wait()
      return c

    return lax.fori_loop(0, BR // HB, grp_body, carry)

  lax.fori_loop(0, NBIG, big_body, 0)
  plsc.subcore_barrier()
  pltpu.sync_copy(accin_sh.at[pl.ds(sid * OROWS, OROWS)],
                  out_hbm.at[cid, 0, pl.ds(sid * OROWS, OROWS)])
  pltpu.sync_copy(accout_sh.at[pl.ds(sid * OROWS, OROWS)],
                  out_hbm.at[cid, 1, pl.ds(sid * OROWS, OROWS)])


NR = NPAD // 128
PR16 = NPAD * 16 // 128
PR8 = NPAD * 8 // 128


def _mask16(x):
  r = lax.broadcasted_iota(jnp.int32, x.shape, 0)
  l = lax.broadcasted_iota(jnp.int32, x.shape, 1)
  return jnp.where(8 * r + l // 16 < N, x, 0.0)


def _mask8(x):
  r = lax.broadcasted_iota(jnp.int32, x.shape, 0)
  l = lax.broadcasted_iota(jnp.int32, x.shape, 1)
  return jnp.where(16 * r + l // 8 < N, x, 0.0)


def _iota2(shape):
  a = lax.broadcasted_iota(jnp.int32, shape, 0)
  b = lax.broadcasted_iota(jnp.int32, shape, 1)
  return a, b


def _kron16(w):
  a, b = _iota2((128, 128))
  return jnp.where(a // 16 == b // 16, jnp.tile(w, (8, 8)), 0.0)


def _kron8(w):
  a, b = _iota2((128, 128))
  return jnp.where(a // 8 == b // 8, jnp.tile(w, (16, 16)), 0.0)


def _dot(a, b):
  return jnp.dot(a, b, preferred_element_type=jnp.float32,
                 precision=lax.Precision.HIGHEST)


def _stage0(degp_ref, x1_ref, nd_ref, ndp16_ref, nsp16_ref,
            ndp8_ref, nsp8_ref):
  din = degp_ref[0, 0] + degp_ref[1, 0]
  dout = degp_ref[0, 1] + degp_ref[1, 1]
  p, l = _iota2((1024, 128))
  r8c0 = jnp.where((p // 8 == l) & (p % 8 == 0), 1.0, 0.0)
  def tonm(x):
    return _dot(jnp.reshape(x, (NR, 1024)), r8c0)
  in_nm = tonm(din)
  out_nm = tonm(dout)
  ns = lax.rsqrt(jnp.maximum(out_nm, 1.0))
  nd = lax.rsqrt(jnp.maximum(in_nm, 1.0))
  l2, q = _iota2((128, 2048))
  e16 = jnp.where(q // 16 == l2, 1.0, 0.0)
  ndp16_ref[...] = jnp.reshape(_dot(nd, e16), (PR16, 128))
  nsp16_ref[...] = jnp.reshape(_dot(ns, e16), (PR16, 128))
  l3, q8 = _iota2((128, 1024))
  e8 = jnp.where(q8 // 8 == l3, 1.0, 0.0)
  ndp8_ref[...] = jnp.reshape(_dot(nd, e8), (PR8, 128))
  nsp8_ref[...] = jnp.reshape(_dot(ns, e8), (PR8, 128))
  nd_ref[...] = nd
  r = lax.broadcasted_iota(jnp.int32, (NR, 128), 0)
  lm = lax.broadcasted_iota(jnp.int32, (NR, 128), 1)
  x1 = jnp.where(128 * r + lm < N, in_nm * ns, 0.0)
  e8c0 = jnp.where((q8 // 8 == l3) & (q8 % 8 == 0), 1.0, 0.0)
  x1_ref[...] = jnp.reshape(_dot(x1, e8c0), (PR8, 128))


def _stage1(mp_ref, nd_ref, nsp_ref, w1_ref, b1_ref, x2_ref):
  m8 = mp_ref[0] + mp_ref[1]
  p, l = _iota2((1024, 128))
  r8c0 = jnp.where((p // 8 == l) & (p % 8 == 0), 1.0, 0.0)
  m_nm = _dot(jnp.reshape(m8, (NR, 1024)), r8c0) * nd_ref[...]
  l2, q = _iota2((128, 2048))
  e16w1 = jnp.where(q // 16 == l2, 1.0, 0.0) * jnp.tile(w1_ref[...],
                                                        (128, 128))
  h = jnp.reshape(_dot(m_nm, e16w1), (PR16, 128))
  b1 = jnp.tile(b1_ref[...], (1, 8))
  x2_ref[...] = _mask16(jnp.maximum(h + b1, 0.0) * nsp_ref[...])


def _stage2(mp_ref, ndp_ref, nsp16_ref, w2_ref, b2_ref, w3_ref, x3_ref):
  m = (mp_ref[0] + mp_ref[1]) * ndp_ref[...]
  b2 = jnp.tile(b2_ref[...], (1, 8))
  h = jnp.maximum(_dot(m, _kron16(w2_ref[...])) + b2, 0.0)
  x3_ref[...] = _mask16(_dot(h, _kron16(w3_ref[...])) * nsp16_ref[...])


def _stage3(mp_ref, ndp8_ref, nsp8_ref, b3_ref, w4_ref, x4_ref):
  m = (mp_ref[0] + mp_ref[1]) * ndp8_ref[...]
  b3 = jnp.tile(b3_ref[...], (1, 16))
  h = jnp.maximum(m + b3, 0.0)
  x4_ref[...] = _mask8(_dot(h, _kron8(w4_ref[...])) * nsp8_ref[...])


def _stage4(mp_ref, ndp8_ref, b4_ref, wc_ref, bc_ref, out_ref):
  m = (mp_ref[0] + mp_ref[1]) * ndp8_ref[...]
  b4 = jnp.tile(b4_ref[...], (1, 16))
  h = _mask8(jnp.maximum(m + b4, 0.0))
  lane = jnp.sum(h, axis=0, keepdims=True)
  l, j = _iota2((128, 8))
  sel = jnp.where(l % 8 == j, 1.0, 0.0)
  g = _dot(lane, sel) * (1.0 / N)
  out_ref[...] = _dot(g[:, :4], wc_ref[...]) + bc_ref[...]


def _tc(body, out_shape, *args):
  return pl.pallas_call(body, out_shape=out_shape)(*args)


def _f32(*shape):
  return jax.ShapeDtypeStruct(shape, jnp.float32)


def kernel(edge_index, num_nodes, W1, b1, W2, b2, W3, b3, W4, b4, Wc, bc):
  del num_nodes
  pad = jnp.full((EPAD - E,), SENT, jnp.int32)
  src2 = jnp.concatenate([edge_index[0], pad]).reshape(EROWS, 128)
  dst2 = jnp.concatenate([edge_index[1], pad]).reshape(EROWS, 128)
  src2x2 = src2 * 2
  ones8 = jnp.ones((128, 8), jnp.float32)
  z8 = jnp.zeros((NPAD, 8), jnp.float32)
  z16 = jnp.zeros((NPAD, 16), jnp.float32)

  degp = _degrees(src2, dst2, ones8, z8)
  x1p, nd, ndp16, nsp16, ndp8, nsp8 = _tc(
      _stage0,
      (_f32(PR8, 128), _f32(NR, 128), _f32(PR16, 128), _f32(PR16, 128),
       _f32(PR8, 128), _f32(PR8, 128)),
      degp.reshape(NC, 2, PR8, 128))
  m1 = _prop8(x1p.reshape(NPAD, 8), src2, dst2, z8)
  x2p = _tc(_stage1, _f32(PR16, 128), m1.reshape(NC, PR8, 128), nd, nsp16,
            W1, b1.reshape(1, 16))
  m2 = _prop16(x2p.reshape(NPAD, 16), src2, dst2, z16)
  w3p = jnp.concatenate([W3, jnp.zeros((16, 8), jnp.float32)], axis=1)
  x3p = _tc(_stage2, _f32(PR16, 128), m2.reshape(NC, PR16, 128), ndp16,
            nsp16, W2, b2.reshape(1, 16), w3p)
  m3 = _prop8(x3p.reshape(2 * NPAD, 8), src2x2, dst2, z8)
  w4p = jnp.concatenate([W4, jnp.zeros((8, 4), jnp.float32)], axis=1)
  x4p = _tc(_stage3, _f32(PR8, 128), m3.reshape(NC, PR8, 128), ndp8, nsp8,
            b3.reshape(1, 8), w4p)
  m4 = _prop8(x4p.reshape(NPAD, 8), src2, dst2, z8)
  b4p = jnp.concatenate([b4, jnp.zeros((4,), jnp.float32)]).reshape(1, 8)
  out = _tc(_stage4, _f32(1, 10), m4.reshape(NC, PR8, 128), ndp8,
            b4p, Wc, bc.reshape(1, 10))
  return out
